# baseline (speedup 1.0000x reference)
import functools

import jax
import jax.numpy as jnp
from jax import lax
from jax.experimental import pallas as pl
from jax.experimental.pallas import tpu as pltpu
from jax.experimental.pallas import tpu_sc as plsc

BATCH = 16384
EMBED = 64
NYEAR = 100000
NGENRE = 26
OUTD = 2 * EMBED
LANES = 16


@functools.cache
def _make_sc_gather():
    info = plsc.get_sparse_core_info()
    nc, ns = info.num_cores, info.num_subcores
    nw = nc * ns
    bpw = BATCH // nw
    mesh = plsc.VectorSubcoreMesh(core_axis_name="c", subcore_axis_name="s")

    @functools.partial(
        pl.kernel,
        mesh=mesh,
        out_type=jax.ShapeDtypeStruct((BATCH * EMBED,), jnp.float32),
        scratch_types=[
            pltpu.VMEM((bpw,), jnp.int32),
            pltpu.VMEM((bpw * EMBED,), jnp.float32),
            pltpu.SemaphoreType.DMA,
        ],
        compiler_params=pltpu.CompilerParams(
            use_tc_tiling_on_sc=False, needs_layout_passes=False),
    )
    def sc_gather(table_hbm, idx_hbm, out_hbm, idx_v, rows_v, sem):
        wid = lax.axis_index("s") * nc + lax.axis_index("c")
        base = wid * bpw
        pltpu.sync_copy(idx_hbm.at[pl.ds(base, bpw)], idx_v)

        def chunk(k, carry):
            k16 = k * LANES
            iv = idx_v[pl.ds(k16, LANES)] * EMBED
            for t in range(LANES):
                src = pl.multiple_of(iv[t], EMBED)
                dst = pl.multiple_of((k16 + t) * EMBED, EMBED)
                pltpu.async_copy(
                    table_hbm.at[pl.ds(src, EMBED)],
                    rows_v.at[pl.ds(dst, EMBED)], sem)
            return carry

        lax.fori_loop(0, bpw // LANES, chunk, 0)
        pltpu.make_async_copy(
            table_hbm.at[pl.ds(0, bpw * EMBED)], rows_v, sem).wait()
        pltpu.sync_copy(rows_v, out_hbm.at[pl.ds(base * EMBED, bpw * EMBED)])

    return sc_gather


def _combine_body(year_ref, g_ref, w_ref, out_ref):
    gf = g_ref[...].astype(jnp.float32)
    s = jnp.sum(gf, axis=1, keepdims=True)
    proj = jax.lax.dot_general(
        gf, w_ref[...], (((1,), (1,)), ((), ())),
        preferred_element_type=jnp.float32)
    out_ref[:, :EMBED] = year_ref[...]
    out_ref[:, EMBED:] = proj / s


def _combine(year, g, w):
    grid = 8
    bs = BATCH // grid
    return pl.pallas_call(
        _combine_body,
        grid=(grid,),
        in_specs=[
            pl.BlockSpec((bs, EMBED), lambda i: (i, 0)),
            pl.BlockSpec((bs, NGENRE), lambda i: (i, 0)),
            pl.BlockSpec((EMBED, NGENRE), lambda i: (0, 0)),
        ],
        out_specs=pl.BlockSpec((bs, OUTD), lambda i: (i, 0)),
        out_shape=jax.ShapeDtypeStruct((BATCH, OUTD), jnp.float32),
    )(year, g, w)


def kernel(year_idx, genre_idx, embedding_year, W_genre):
    idx = year_idx.astype(jnp.int32)
    year_flat = _make_sc_gather()(embedding_year.reshape(-1), idx)
    return _combine(year_flat.reshape(BATCH, EMBED), genre_idx, W_genre)


# speedup vs baseline: 1.4478x; 1.4478x over previous
import functools

import jax
import jax.numpy as jnp
from jax import lax
from jax.experimental import pallas as pl
from jax.experimental.pallas import tpu as pltpu
from jax.experimental.pallas import tpu_sc as plsc

BATCH = 16384
EMBED = 64
NGENRE = 26
OUTD = 2 * EMBED
LANES = 16


@functools.cache
def _make_sc_gather():
    info = plsc.get_sparse_core_info()
    nc, ns = info.num_cores, info.num_subcores
    nw = nc * ns
    bpw = BATCH // nw
    mesh = plsc.VectorSubcoreMesh(core_axis_name="c", subcore_axis_name="s")

    @functools.partial(
        pl.kernel,
        mesh=mesh,
        out_type=jax.ShapeDtypeStruct((BATCH, EMBED), jnp.float32),
        scratch_types=[
            pltpu.VMEM((bpw,), jnp.int32),
            pltpu.VMEM((bpw, EMBED), jnp.float32),
            pltpu.SemaphoreType.DMA,
        ],
        compiler_params=pltpu.CompilerParams(use_tc_tiling_on_sc=True),
    )
    def sc_gather(table_hbm, idx_hbm, out_hbm, idx_v, rows_v, sem):
        wid = lax.axis_index("s") * nc + lax.axis_index("c")
        base = wid * bpw
        pltpu.sync_copy(idx_hbm.at[pl.ds(base, bpw)], idx_v)

        def chunk(k, carry):
            k16 = k * LANES
            iv = idx_v[pl.ds(k16, LANES)]
            for t in range(LANES):
                pltpu.async_copy(
                    table_hbm.at[iv[t]], rows_v.at[k16 + t], sem)
            return carry

        lax.fori_loop(0, bpw // LANES, chunk, 0)
        pltpu.make_async_copy(
            table_hbm.at[pl.ds(0, bpw)], rows_v, sem).wait()
        pltpu.sync_copy(rows_v, out_hbm.at[pl.ds(base, bpw)])

    return sc_gather


def _combine_body(year_ref, g_ref, w_ref, out_ref):
    gf = g_ref[...].astype(jnp.float32)
    s = jnp.sum(gf, axis=1, keepdims=True)
    proj = jax.lax.dot_general(
        gf, w_ref[...], (((1,), (1,)), ((), ())),
        preferred_element_type=jnp.float32)
    out_ref[:, :EMBED] = year_ref[...]
    out_ref[:, EMBED:] = proj / s


def _combine(year, g, w):
    grid = 8
    bs = BATCH // grid
    return pl.pallas_call(
        _combine_body,
        grid=(grid,),
        in_specs=[
            pl.BlockSpec((bs, EMBED), lambda i: (i, 0)),
            pl.BlockSpec((bs, NGENRE), lambda i: (i, 0)),
            pl.BlockSpec((EMBED, NGENRE), lambda i: (0, 0)),
        ],
        out_specs=pl.BlockSpec((bs, OUTD), lambda i: (i, 0)),
        out_shape=jax.ShapeDtypeStruct((BATCH, OUTD), jnp.float32),
    )(year, g, w)


def kernel(year_idx, genre_idx, embedding_year, W_genre):
    idx = year_idx.astype(jnp.int32)
    year_emb = _make_sc_gather()(embedding_year, idx)
    return _combine(year_emb, genre_idx, W_genre)
